# baseline (speedup 1.0000x reference)
import jax
import jax.numpy as jnp
from jax.experimental import pallas as pl
from jax.experimental.pallas import tpu as pltpu

N = 8192
M = 8192
BJ = 512


def _chamfer_kernel(x_ref, yt_ref, out_ref, xm_ref, xn_ref, rowmin_ref,
                    colacc_ref):
    j = pl.program_id(0)
    nj = pl.num_programs(0)

    @pl.when(j == 0)
    def _init():
        x = x_ref[...]
        xm_ref[...] = x * -2.0
        xn_ref[...] = jnp.sum(x * x, axis=1, keepdims=True)
        rowmin_ref[...] = jnp.full_like(rowmin_ref, jnp.inf)
        colacc_ref[0, 0] = 0.0

    yt = yt_ref[...]
    yn = jnp.sum(yt * yt, axis=0, keepdims=True)
    d = (xn_ref[...] + yn) + jnp.dot(xm_ref[...], yt)

    rowmin_ref[...] = jnp.minimum(rowmin_ref[...], jnp.min(d, axis=1, keepdims=True))
    colacc_ref[0, 0] += jnp.sum(jnp.min(d, axis=0))

    @pl.when(j == nj - 1)
    def _finish():
        mean_row = jnp.sum(rowmin_ref[...]) / N
        mean_col = colacc_ref[0, 0] / M
        out_ref[...] = jnp.full((1, 1), (mean_row + mean_col) * 0.5, jnp.float32)


def kernel(pred_positions, target_positions):
    yt = target_positions.T
    out = pl.pallas_call(
        _chamfer_kernel,
        grid=(M // BJ,),
        in_specs=[
            pl.BlockSpec((N, 3), lambda j: (0, 0)),
            pl.BlockSpec((3, BJ), lambda j: (0, j)),
        ],
        out_specs=pl.BlockSpec((1, 1), lambda j: (0, 0)),
        out_shape=jax.ShapeDtypeStruct((1, 1), jnp.float32),
        scratch_shapes=[
            pltpu.VMEM((N, 3), jnp.float32),
            pltpu.VMEM((N, 1), jnp.float32),
            pltpu.VMEM((N, 1), jnp.float32),
            pltpu.SMEM((1, 1), jnp.float32),
        ],
    )(pred_positions, yt)
    return out[0, 0]


# speedup vs baseline: 1.2635x; 1.2635x over previous
import jax
import jax.numpy as jnp
from jax.experimental import pallas as pl
from jax.experimental.pallas import tpu as pltpu

N = 8192
M = 8192
BI = 4096
BJ = 2048
NI = N // BI
NJ = M // BJ


def _chamfer_kernel(x_ref, yt_ref, out_ref, xm_ref, xn_ref, rowmin_ref,
                    colmin_ref):
    i = pl.program_id(0)
    j = pl.program_id(1)

    @pl.when(jnp.logical_and(i == 0, j == 0))
    def _init():
        x = x_ref[...]
        xm_ref[...] = x * -2.0
        xn_ref[...] = jnp.sum(x * x, axis=1, keepdims=True)

    yt = yt_ref[...]
    yn = jnp.sum(yt * yt, axis=0, keepdims=True)
    xm = xm_ref[pl.ds(i * BI, BI), :]
    xn = xn_ref[pl.ds(i * BI, BI), :]
    d = (xn + yn) + jnp.dot(xm, yt)

    rpart = jnp.min(d, axis=1, keepdims=True)
    cpart = jnp.min(d, axis=0, keepdims=True)

    @pl.when(j == 0)
    def _row_first():
        rowmin_ref[pl.ds(i * BI, BI), :] = rpart

    @pl.when(j > 0)
    def _row_acc():
        rowmin_ref[pl.ds(i * BI, BI), :] = jnp.minimum(
            rowmin_ref[pl.ds(i * BI, BI), :], rpart)

    @pl.when(i == 0)
    def _col_first():
        colmin_ref[:, pl.ds(j * BJ, BJ)] = cpart

    @pl.when(i > 0)
    def _col_acc():
        colmin_ref[:, pl.ds(j * BJ, BJ)] = jnp.minimum(
            colmin_ref[:, pl.ds(j * BJ, BJ)], cpart)

    @pl.when(jnp.logical_and(i == NI - 1, j == NJ - 1))
    def _finish():
        mean_row = jnp.sum(rowmin_ref[...]) / N
        mean_col = jnp.sum(colmin_ref[...]) / M
        out_ref[...] = jnp.full((1, 1), (mean_row + mean_col) * 0.5, jnp.float32)


def kernel(pred_positions, target_positions):
    yt = target_positions.T
    out = pl.pallas_call(
        _chamfer_kernel,
        grid=(NI, NJ),
        in_specs=[
            pl.BlockSpec((N, 3), lambda i, j: (0, 0)),
            pl.BlockSpec((3, BJ), lambda i, j: (0, j)),
        ],
        out_specs=pl.BlockSpec((1, 1), lambda i, j: (0, 0)),
        out_shape=jax.ShapeDtypeStruct((1, 1), jnp.float32),
        scratch_shapes=[
            pltpu.VMEM((N, 3), jnp.float32),
            pltpu.VMEM((N, 1), jnp.float32),
            pltpu.VMEM((N, 1), jnp.float32),
            pltpu.VMEM((1, M), jnp.float32),
        ],
    )(pred_positions, yt)
    return out[0, 0]
